# Initial kernel scaffold; baseline (speedup 1.0000x reference)
import jax
import jax.numpy as jnp
from jax.experimental import pallas as pl


def kernel(origins, directions, lengths, xys, ray_weights):
    raise NotImplementedError("write your pallas kernel here")


# speedup vs baseline: 4.1657x; 4.1657x over previous
import functools

import jax
import jax.numpy as jnp
from jax import lax
from jax.experimental import pallas as pl
from jax.experimental.pallas import tpu as pltpu
from jax.experimental.pallas import tpu_sc as plsc

EPS = 1e-5
LANES = 16
NUM_WORKERS = 32
P = 64
NS = 64
OUT_P = P + NS


def _refine_body(lengths_hbm, weights_hbm, out_hbm, ltile, wtile, cdft, belowt,
                 zt, outtile):
    num_rays = lengths_hbm.shape[0]
    rays_per_worker = num_rays // NUM_WORKERS
    batches = rays_per_worker // LANES

    wid = lax.axis_index("s") * 2 + lax.axis_index("c")
    lane = lax.iota(jnp.int32, LANES)
    zero_f = jnp.zeros((LANES,), jnp.float32)
    zero_i = jnp.zeros((LANES,), jnp.int32)

    def batch_body(b, _):
        base = wid * rays_per_worker + b * LANES
        pltpu.sync_copy(lengths_hbm.at[pl.ds(base, LANES)], ltile)
        pltpu.sync_copy(weights_hbm.at[pl.ds(base, LANES)], wtile)

        cdft[0] = zero_f

        def cdf_body(k, run):
            w = plsc.load_gather(wtile, [lane, jnp.full((LANES,), k + 1, jnp.int32)])
            run = run + (w + EPS)
            cdft[k + 1] = run
            return run

        total = lax.fori_loop(0, P - 2, cdf_body, zero_f)
        inv = (NS - 1.0) / total

        def init_body(j, c):
            belowt[j] = zero_i
            return c

        lax.fori_loop(0, NS, init_body, 0)

        def pos_body(k, c):
            x = cdft[k] * inv
            i = x.astype(jnp.int32)
            i = jnp.where(i.astype(jnp.float32) < x, i + 1, i)
            p = jnp.clip(i, 0, NS - 1)
            plsc.store_scatter(belowt, [p, lane], jnp.full((LANES,), k, jnp.int32))
            return c

        lax.fori_loop(0, P - 1, pos_body, 0)

        def j_body(j, runb):
            runb = jnp.maximum(runb, belowt[j])
            bi = runb
            ai = jnp.minimum(bi + 1, P - 2)
            cb = plsc.load_gather(cdft, [bi, lane])
            ca = plsc.load_gather(cdft, [ai, lane])
            lb0 = plsc.load_gather(ltile, [lane, bi])
            lb1 = plsc.load_gather(ltile, [lane, bi + 1])
            la0 = plsc.load_gather(ltile, [lane, ai])
            la1 = plsc.load_gather(ltile, [lane, ai + 1])
            bins_b = 0.5 * (lb0 + lb1)
            bins_a = 0.5 * (la0 + la1)
            u = lax.convert_element_type(j, jnp.float32) * (1.0 / (NS - 1.0)) * total
            den = ca - cb
            safe = jnp.where(den < EPS * total, total, den)
            t = (u - cb) / safe
            zt[j] = bins_b + t * (bins_a - bins_b)
            return runb

        lax.fori_loop(0, NS, j_body, zero_i)

        def m_body(i, carry):
            pa, pb = carry
            av = plsc.load_gather(ltile, [lane, jnp.minimum(pa, P - 1)])
            bv = plsc.load_gather(zt, [jnp.minimum(pb, NS - 1), lane])
            take_a = (pb > NS - 1) | ((pa <= P - 1) & (av <= bv))
            v = jnp.where(take_a, av, bv)
            plsc.store_scatter(outtile, [lane, jnp.full((LANES,), i, jnp.int32)], v)
            pa = pa + jnp.where(take_a, 1, 0)
            pb = pb + jnp.where(take_a, 0, 1)
            return (pa, pb)

        lax.fori_loop(0, OUT_P, m_body, (zero_i, zero_i))

        pltpu.sync_copy(outtile, out_hbm.at[pl.ds(base, LANES)])
        return _

    lax.fori_loop(0, batches, batch_body, 0)


@jax.jit
def _refine(lengths2d, weights2d):
    num_rays = lengths2d.shape[0]
    mesh = plsc.VectorSubcoreMesh(core_axis_name="c", subcore_axis_name="s")
    return pl.kernel(
        _refine_body,
        out_type=jax.ShapeDtypeStruct((num_rays, OUT_P), jnp.float32),
        mesh=mesh,
        compiler_params=pltpu.CompilerParams(needs_layout_passes=False),
        scratch_types=[
            pltpu.VMEM((LANES, P), jnp.float32),
            pltpu.VMEM((LANES, P), jnp.float32),
            pltpu.VMEM((P - 1, LANES), jnp.float32),
            pltpu.VMEM((NS, LANES), jnp.int32),
            pltpu.VMEM((NS, LANES), jnp.float32),
            pltpu.VMEM((LANES, OUT_P), jnp.float32),
        ],
    )(lengths2d, weights2d)


def kernel(origins, directions, lengths, xys, ray_weights):
    b, r, p = lengths.shape
    z_out = _refine(lengths.reshape(b * r, p), ray_weights.reshape(b * r, p))
    return (origins, directions, z_out.reshape(b, r, OUT_P), xys)


# speedup vs baseline: 5.3473x; 1.2837x over previous
import functools

import jax
import jax.numpy as jnp
from jax import lax
from jax.experimental import pallas as pl
from jax.experimental.pallas import tpu as pltpu
from jax.experimental.pallas import tpu_sc as plsc

EPS = 1e-5
LANES = 16
NUM_WORKERS = 32
G = 4
BATCH = G * LANES
P = 64
NS = 64
OUT_P = P + NS


def _compute_batch(lt, wt, ot, cdfts, binsts, belowts, zts, lane,
                   pre_merge=None):
    lanes = [lane + (LANES * g) for g in range(G)]
    zero_f = jnp.zeros((LANES,), jnp.float32)
    zero_i = jnp.zeros((LANES,), jnp.int32)

    for g in range(G):
        cdfts[g][0] = zero_f

    def cdf_body(k, carry):
        runs, prevs = carry
        kv = jnp.full((LANES,), k, jnp.int32)
        new_runs, new_prevs = [], []
        for g in range(G):
            w = plsc.load_gather(wt, [lanes[g], kv + 1])
            lcol = plsc.load_gather(lt, [lanes[g], kv + 1])
            r = runs[g] + (w + EPS)
            cdfts[g][k + 1] = r
            binsts[g][k] = 0.5 * (prevs[g] + lcol)
            new_runs.append(r)
            new_prevs.append(lcol)
        return tuple(new_runs), tuple(new_prevs)

    prev0 = tuple(plsc.load_gather(lt, [lanes[g], zero_i]) for g in range(G))
    totals, prevs = lax.fori_loop(0, P - 2, cdf_body, ((zero_f,) * G, prev0))
    kv62 = jnp.full((LANES,), P - 1, jnp.int32)
    for g in range(G):
        lcol = plsc.load_gather(lt, [lanes[g], kv62])
        binsts[g][P - 2] = 0.5 * (prevs[g] + lcol)

    invs = [(NS - 1.0) / totals[g] for g in range(G)]

    def init_body(j, c):
        for g in range(G):
            belowts[g][j] = zero_i
        return c

    lax.fori_loop(0, NS, init_body, 0)

    def pos_body(k, c):
        kv = jnp.full((LANES,), k, jnp.int32)
        for g in range(G):
            x = cdfts[g][k] * invs[g]
            i = x.astype(jnp.int32)
            i = jnp.where(i.astype(jnp.float32) < x, i + 1, i)
            p = jnp.clip(i, 0, NS - 1)
            plsc.store_scatter(belowts[g], [p, lane], kv)
        return c

    lax.fori_loop(0, P - 1, pos_body, 0)

    def j_body(j, runbs):
        uf = lax.convert_element_type(j, jnp.float32) * (1.0 / (NS - 1.0))
        out = []
        for g in range(G):
            runb = jnp.maximum(runbs[g], belowts[g][j])
            bi = runb
            ai = jnp.minimum(bi + 1, P - 2)
            cb = plsc.load_gather(cdfts[g], [bi, lane])
            ca = plsc.load_gather(cdfts[g], [ai, lane])
            bb = plsc.load_gather(binsts[g], [bi, lane])
            ba = plsc.load_gather(binsts[g], [ai, lane])
            u = uf * totals[g]
            den = ca - cb
            safe = jnp.where(den < EPS * totals[g], totals[g], den)
            t = (u - cb) / safe
            zts[g][j] = bb + t * (ba - bb)
            out.append(runb)
        return tuple(out)

    lax.fori_loop(0, NS, j_body, (zero_i,) * G)

    if pre_merge is not None:
        pre_merge()

    def m_body(i, carry):
        pas, pbs = carry
        iv = jnp.full((LANES,), i, jnp.int32)
        new_pas, new_pbs = [], []
        for g in range(G):
            pa, pb = pas[g], pbs[g]
            av = plsc.load_gather(lt, [lanes[g], jnp.minimum(pa, P - 1)])
            bv = plsc.load_gather(zts[g], [jnp.minimum(pb, NS - 1), lane])
            take_a = (pb > NS - 1) | ((pa <= P - 1) & (av <= bv))
            v = jnp.where(take_a, av, bv)
            plsc.store_scatter(ot, [lanes[g], iv], v)
            new_pas.append(pa + jnp.where(take_a, 1, 0))
            new_pbs.append(pb + jnp.where(take_a, 0, 1))
        return tuple(new_pas), tuple(new_pbs)

    lax.fori_loop(0, OUT_P, m_body, ((zero_i,) * G, (zero_i,) * G))


def _refine_body(lengths_hbm, weights_hbm, out_hbm, lt0, lt1, wt0, wt1,
                 ot,
                 cdft0, cdft1, cdft2, cdft3,
                 binst0, binst1, binst2, binst3,
                 belowt0, belowt1, belowt2, belowt3,
                 zt0, zt1, zt2, zt3,
                 sl0, sl1, sw0, sw1, so):
    cdfts = [cdft0, cdft1, cdft2, cdft3]
    binsts = [binst0, binst1, binst2, binst3]
    belowts = [belowt0, belowt1, belowt2, belowt3]
    zts = [zt0, zt1, zt2, zt3]
    num_rays = lengths_hbm.shape[0]
    rays_per_worker = num_rays // NUM_WORKERS
    nb = rays_per_worker // BATCH

    wid = lax.axis_index("s") * 2 + lax.axis_index("c")
    w_base = wid * rays_per_worker
    lane = lax.iota(jnp.int32, LANES)

    def in_l(slot_ref, sem, b):
        return pltpu.make_async_copy(
            lengths_hbm.at[pl.ds(w_base + b * BATCH, BATCH)], slot_ref, sem)

    def in_w(slot_ref, sem, b):
        return pltpu.make_async_copy(
            weights_hbm.at[pl.ds(w_base + b * BATCH, BATCH)], slot_ref, sem)

    def out_c(slot_ref, sem, b):
        return pltpu.make_async_copy(
            slot_ref, out_hbm.at[pl.ds(w_base + b * BATCH, BATCH)], sem)

    in_l(lt0, sl0, 0).start()
    in_w(wt0, sw0, 0).start()

    def wait_out():
        out_c(ot, so, 0).wait()

    def pair_body(b2, c):
        e = b2 * 2
        o = e + 1
        in_l(lt0, sl0, e).wait()
        in_w(wt0, sw0, e).wait()
        in_l(lt1, sl1, o).start()
        in_w(wt1, sw1, o).start()

        def pre_merge_e():
            @pl.when(b2 > 0)
            def _():
                wait_out()

        _compute_batch(lt0, wt0, ot, cdfts, binsts, belowts, zts, lane,
                       pre_merge=pre_merge_e)
        out_c(ot, so, e).start()

        in_l(lt1, sl1, o).wait()
        in_w(wt1, sw1, o).wait()

        @pl.when(b2 < nb // 2 - 1)
        def _():
            in_l(lt0, sl0, o + 1).start()
            in_w(wt0, sw0, o + 1).start()

        _compute_batch(lt1, wt1, ot, cdfts, binsts, belowts, zts, lane,
                       pre_merge=wait_out)
        out_c(ot, so, o).start()
        return c

    lax.fori_loop(0, nb // 2, pair_body, 0)
    wait_out()


@jax.jit
def _refine(lengths2d, weights2d):
    num_rays = lengths2d.shape[0]
    mesh = plsc.VectorSubcoreMesh(core_axis_name="c", subcore_axis_name="s")
    return pl.kernel(
        _refine_body,
        out_type=jax.ShapeDtypeStruct((num_rays, OUT_P), jnp.float32),
        mesh=mesh,
        compiler_params=pltpu.CompilerParams(
            needs_layout_passes=False, use_tc_tiling_on_sc=False),
        scratch_types=[
            pltpu.VMEM((BATCH, P), jnp.float32),
            pltpu.VMEM((BATCH, P), jnp.float32),
            pltpu.VMEM((BATCH, P), jnp.float32),
            pltpu.VMEM((BATCH, P), jnp.float32),
            pltpu.VMEM((BATCH, OUT_P), jnp.float32),
            *[pltpu.VMEM((P - 1, LANES), jnp.float32) for _ in range(G)],
            *[pltpu.VMEM((P - 1, LANES), jnp.float32) for _ in range(G)],
            *[pltpu.VMEM((NS, LANES), jnp.int32) for _ in range(G)],
            *[pltpu.VMEM((NS, LANES), jnp.float32) for _ in range(G)],
            pltpu.SemaphoreType.DMA,
            pltpu.SemaphoreType.DMA,
            pltpu.SemaphoreType.DMA,
            pltpu.SemaphoreType.DMA,
            pltpu.SemaphoreType.DMA,
        ],
    )(lengths2d, weights2d)


def kernel(origins, directions, lengths, xys, ray_weights):
    b, r, p = lengths.shape
    z_out = _refine(lengths.reshape(b * r, p), ray_weights.reshape(b * r, p))
    return (origins, directions, z_out.reshape(b, r, OUT_P), xys)
